# Initial kernel scaffold; baseline (speedup 1.0000x reference)
import jax
import jax.numpy as jnp
from jax.experimental import pallas as pl


def kernel(x, batch, ei, W):
    raise NotImplementedError("write your pallas kernel here")


# speedup vs baseline: 22.9565x; 22.9565x over previous
import dataclasses
import functools

import jax
import jax.numpy as jnp
from jax import lax
from jax.experimental import pallas as pl
from jax.experimental.pallas import tpu as pltpu
from jax.experimental.pallas import tpu_sc as plsc

N = 10000
E = 320000
FIN = 128
FQK = 128
FV = 128
H = 8
FH = 16
NTILES = 32
EPT = E // NTILES
EB = 16
STEPS = EPT // EB
NP = 10240
ND = NP // 8
RPT = NP // 16
DPT = ND // 16
ZB = 16


def _proj_body(x_ref, w_ref, q_ref, kv_ref):
    p = lax.dot_general(x_ref[...], w_ref[...], (((1,), (1,)), ((), ())),
                        preferred_element_type=jnp.float32)
    q_ref[...] = p[:, :FQK] * (FH ** -0.5)
    kv_ref[...] = p[:, FQK:]


def _project(x, W):
    blk = 1000
    grid = (N // blk,)
    return pl.pallas_call(
        _proj_body,
        grid=grid,
        in_specs=[
            pl.BlockSpec((blk, FIN), lambda i: (i, 0)),
            pl.BlockSpec((2 * FQK + FV, FIN), lambda i: (0, 0)),
        ],
        out_specs=[
            pl.BlockSpec((blk, FQK), lambda i: (i, 0)),
            pl.BlockSpec((blk, FQK + FV), lambda i: (i, 0)),
        ],
        out_shape=[
            jax.ShapeDtypeStruct((N, FQK), jnp.float32),
            jax.ShapeDtypeStruct((N, FQK + FV), jnp.float32),
        ],
    )(x, W)


def _sc_body(q_hbm, kv_hbm, src_hbm, dst_hbm, num_hbm, den_hbm,
             srcv, dstv, didx, qv, kvv, wvv, dnv, zv, acc_n, acc_d):
    cid = lax.axis_index("c")
    sid = lax.axis_index("s")
    wid = cid * 16 + sid
    lanes = lax.broadcasted_iota(jnp.int32, (16,), 0)
    zero16 = jnp.zeros((16,), jnp.float32)

    @pl.loop(0, ZB)
    def _(i):
        for j in range(FV // 16):
            zv[i, pl.ds(16 * j, 16)] = zero16

    for r in range(RPT // ZB):
        pltpu.sync_copy(zv, acc_n.at[pl.ds(sid * RPT + r * ZB, ZB)])
    for r in range(DPT // ZB):
        pltpu.sync_copy(zv, acc_d.at[pl.ds(sid * DPT + r * ZB, ZB)])
    plsc.subcore_barrier()

    tile_base = wid * EPT

    @pl.loop(0, STEPS)
    def _(step):
        base = tile_base + step * EB
        pltpu.sync_copy(src_hbm.at[pl.ds(base, EB)], srcv)
        pltpu.sync_copy(dst_hbm.at[pl.ds(base, EB)], dstv)
        pltpu.sync_copy(q_hbm.at[srcv], qv)
        pltpu.sync_copy(kv_hbm.at[dstv], kvv)

        @pl.loop(0, EB // 16)
        def _(c):
            sv = srcv[pl.ds(c * 16, 16)]
            didx[pl.ds(c * 16, 16)] = lax.shift_right_logical(sv, 3)
            grpv = sv & 7
            for l in range(16):
                e = c * 16 + l
                grp = grpv[l]
                den = zero16
                for h in range(H):
                    qh = qv[e, pl.ds(16 * h, 16)]
                    kh = kvv[e, pl.ds(16 * h, 16)]
                    vh = kvv[e, pl.ds(FQK + 16 * h, 16)]
                    s = jnp.sum(qh * kh)
                    w = jnp.exp(lax.broadcast(s, (16,)))
                    wvv[e, pl.ds(16 * h, 16)] = w * vh
                    den = den + jnp.where(lanes == h, w, 0.0)
                for g in range(8):
                    sel = lax.broadcast(grp == g, (16,))
                    dnv[e, pl.ds(16 * g, 16)] = lax.select(sel, den, zero16)

        pltpu.sync_copy(wvv, acc_n.at[srcv], add=True)
        pltpu.sync_copy(dnv, acc_d.at[didx], add=True)

    plsc.subcore_barrier()
    for r in range(RPT // ZB):
        pltpu.sync_copy(acc_n.at[pl.ds(sid * RPT + r * ZB, ZB)], zv)
        pltpu.sync_copy(zv, num_hbm.at[cid, pl.ds(sid * RPT + r * ZB, ZB)])
    for r in range(DPT // ZB):
        pltpu.sync_copy(acc_d.at[pl.ds(sid * DPT + r * ZB, ZB)], zv)
        pltpu.sync_copy(zv, den_hbm.at[cid, pl.ds(sid * DPT + r * ZB, ZB)])


def _sc_attend(q_tbl, kv_tbl, src, dst):
    mesh = plsc.VectorSubcoreMesh(core_axis_name="c", subcore_axis_name="s")
    cp = pltpu.CompilerParams()
    if "needs_layout_passes" in pltpu.CompilerParams.__dataclass_fields__:
        cp = dataclasses.replace(cp, needs_layout_passes=False)
    fn = pl.kernel(
        _sc_body,
        compiler_params=cp,
        out_type=[
            jax.ShapeDtypeStruct((2, NP, FV), jnp.float32),
            jax.ShapeDtypeStruct((2, ND, 128), jnp.float32),
        ],
        mesh=mesh,
        scratch_types=[
            pltpu.VMEM((EB,), jnp.int32),
            pltpu.VMEM((EB,), jnp.int32),
            pltpu.VMEM((EB,), jnp.int32),
            pltpu.VMEM((EB, FQK), jnp.float32),
            pltpu.VMEM((EB, FQK + FV), jnp.float32),
            pltpu.VMEM((EB, FV), jnp.float32),
            pltpu.VMEM((EB, 128), jnp.float32),
            pltpu.VMEM((ZB, 128), jnp.float32),
            pltpu.VMEM_SHARED((NP, FV), jnp.float32),
            pltpu.VMEM_SHARED((ND, 128), jnp.float32),
        ],
    )
    return fn(q_tbl, kv_tbl, src, dst)


def _comb_body(num_ref, den_ref, o_ref):
    num = num_ref[0] + num_ref[1]
    den16 = den_ref[0] + den_ref[1]
    col = lax.broadcasted_iota(jnp.int32, (16, FV), 1) // FH
    row = lax.broadcasted_iota(jnp.int32, (16, FV), 0)
    ex = (col == row).astype(jnp.float32)
    rep = lax.dot_general(den16, ex, (((1,), (0,)), ((), ())),
                          preferred_element_type=jnp.float32)
    o_ref[...] = jnp.where(rep > 0, num / rep, 0.0)


def _combine(nd_num, nd_den16):
    blk = 1000
    return pl.pallas_call(
        _comb_body,
        grid=(N // blk,),
        in_specs=[
            pl.BlockSpec((2, blk, FV), lambda i: (0, i, 0)),
            pl.BlockSpec((2, blk, 16), lambda i: (0, i, 0)),
        ],
        out_specs=pl.BlockSpec((blk, FV), lambda i: (i, 0)),
        out_shape=jax.ShapeDtypeStruct((N, FV), jnp.float32),
    )(nd_num, nd_den16)


def kernel(x, batch, ei, W):
    del batch
    q_tbl, kv_tbl = _project(x, W)
    nd_num, nd_den = _sc_attend(q_tbl, kv_tbl, ei[0], ei[1])
    return _combine(nd_num, nd_den.reshape(2, NP, 16))
